# baseline (speedup 1.0000x reference)
import dataclasses
import functools

import jax
import jax.numpy as jnp
from jax import lax
from jax.experimental import pallas as pl
from jax.experimental.pallas import tpu as pltpu
from jax.experimental.pallas import tpu_sc as plsc

MASK_ID = 3
PAD = 1
D = 768

NUM_CORES = 2
NUM_SUBCORES = 16
NW = NUM_CORES * NUM_SUBCORES
LANES = 16

W_CHUNK = 32

MASK_RATIO_TRAIN = 0.15 * 0.8


def _prep_body(inp_ref, mask_ref, coef_ref, mf_ref, pstart_ref, lpofs_ref):
    m = mask_ref[...]
    inp = inp_ref[...]
    b, s = m.shape
    ism = inp == MASK_ID
    c = m
    d = 1
    while d < s:
        shifted = jnp.concatenate(
            [jnp.zeros((b, d), jnp.int32), c[:, :-d]], axis=1
        )
        c = c + shifted
        d *= 2
    e = c - m
    r = jax.lax.broadcasted_iota(jnp.int32, (b, s), 1) & (W_CHUNK - 1)
    f = jnp.where(r == 0, e, 0)
    d = 1
    while d < W_CHUNK:
        shifted = jnp.concatenate(
            [jnp.zeros((b, d), jnp.int32), f[:, :-d]], axis=1
        )
        f = f + jnp.where(r >= d, shifted, 0)
        d *= 2
    pstart_ref[...] = (f + 2) * D
    lpofs_ref[...] = (e - f + 1) * m * D

    src_len = jnp.sum(m, axis=1, keepdims=True).astype(jnp.float32)
    n_mask = jnp.sum(ism.astype(jnp.int32), axis=1, keepdims=True).astype(
        jnp.float32
    )
    ratio = n_mask / src_len
    scale = jnp.minimum((1.0 - MASK_RATIO_TRAIN) / (1.0 - ratio), 4.0)
    coef_ref[...] = scale * m.astype(jnp.float32) * jnp.where(ism, 0.0, 1.0)
    mf_ref[...] = m.astype(jnp.float32)


def _make_prep(b, s):
    return pl.pallas_call(
        _prep_body,
        out_shape=(
            jax.ShapeDtypeStruct((b, s), jnp.float32),
            jax.ShapeDtypeStruct((b, s), jnp.float32),
            jax.ShapeDtypeStruct((b, s), jnp.int32),
            jax.ShapeDtypeStruct((b, s), jnp.int32),
        ),
    )


def _make_sc_gather(n_tokens):
    per_w = n_tokens // NW
    n_chunks = per_w // W_CHUNK
    p_words = W_CHUNK * D
    pb_words = (W_CHUNK + 1) * D

    mesh = plsc.VectorSubcoreMesh(core_axis_name="c", subcore_axis_name="s")

    cp = pltpu.CompilerParams()
    if "needs_layout_passes" in pltpu.CompilerParams.__dataclass_fields__:
        cp = dataclasses.replace(cp, needs_layout_passes=False)

    @functools.partial(
        pl.kernel,
        out_type=jax.ShapeDtypeStruct((n_tokens, D), jnp.float32),
        mesh=mesh,
        compiler_params=cp,
        scratch_types=[
            pltpu.VMEM((per_w,), jnp.int32),
            pltpu.VMEM((per_w,), jnp.int32),
            pltpu.VMEM((per_w,), jnp.int32),
            pltpu.VMEM((per_w,), jnp.float32),
            pltpu.VMEM((W_CHUNK, D), jnp.float32),
            pltpu.VMEM((W_CHUNK, D), jnp.float32),
            pltpu.VMEM((pb_words,), jnp.float32),
            pltpu.VMEM((pb_words,), jnp.float32),
            pltpu.SemaphoreType.DMA,
            pltpu.SemaphoreType.DMA,
            pltpu.SemaphoreType.DMA,
            pltpu.SemaphoreType.DMA,
            pltpu.SemaphoreType.DMA,
        ],
    )
    def sc_kernel(
        idx_hbm, pstart_hbm, lpofs_hbm, coef_hbm, word_hbm,
        posflat_hbm, out_hbm,
        idx_v, pstart_v, lpofs_v, coef_v, wb0, wb1, pb0, pb1,
        lsem, gs0, gs1, os0, os1,
    ):
        wid = lax.axis_index("s") * NUM_CORES + lax.axis_index("c")
        base = wid * per_w
        sml = (
            pltpu.async_copy(idx_hbm.at[pl.ds(base, per_w)], idx_v, lsem),
            pltpu.async_copy(
                pstart_hbm.at[pl.ds(base, per_w)], pstart_v, lsem
            ),
            pltpu.async_copy(lpofs_hbm.at[pl.ds(base, per_w)], lpofs_v, lsem),
            pltpu.async_copy(coef_hbm.at[pl.ds(base, per_w)], coef_v, lsem),
        )
        zero16 = jnp.zeros((LANES,), jnp.float32)
        for c0 in range(0, D, LANES):
            pb0[pl.ds(c0, LANES)] = zero16
            pb1[pl.ds(c0, LANES)] = zero16
        for c_ in sml:
            c_.wait()

        wb = (wb0, wb1)
        pb = (pb0, pb1)
        gs = (gs0, gs1)
        osem = (os0, os1)
        pend_g = [None, None]
        pend_o = [None, None]

        def issue_fetch(j):
            k = j % 2
            t0 = j * W_CHUNK
            cw = pltpu.async_copy(
                word_hbm.at[idx_v.at[pl.ds(t0, W_CHUNK)]], wb[k], gs[k]
            )
            start = pl.multiple_of(pstart_v[pl.ds(t0, LANES)][0], 8)
            cpos = pltpu.async_copy(
                posflat_hbm.at[pl.ds(start, p_words)],
                pb[k].at[pl.ds(D, p_words)],
                gs[k],
            )
            pend_g[k] = (cw, cpos)

        issue_fetch(0)
        for j in range(n_chunks):
            k = j % 2
            if j + 1 < n_chunks:
                if pend_o[1 - k] is not None:
                    pend_o[1 - k].wait()
                    pend_o[1 - k] = None
                issue_fetch(j + 1)
            for c_ in pend_g[k]:
                c_.wait()
            pend_g[k] = None

            t0 = j * W_CHUNK

            rows = []
            for g in range(W_CHUNK // LANES):
                tg = t0 + g * LANES
                lp16 = lpofs_v[pl.ds(tg, LANES)]
                co16 = coef_v[pl.ds(tg, LANES)]
                for r in range(LANES):
                    rows.append((g * LANES + r, lp16[r], co16[r]))

            @plsc.parallel_loop(0, D, LANES)
            def _(c0):
                for tok, lp, co in rows:
                    w = wb[k].at[tok, pl.ds(c0, LANES)][...]
                    p = pb[k][pl.ds(pl.multiple_of(lp + c0, 8), LANES)]
                    wb[k].at[tok, pl.ds(c0, LANES)][...] = w * co + p

            pend_o[k] = pltpu.async_copy(
                wb[k], out_hbm.at[pl.ds(base + t0, W_CHUNK)], osem[k]
            )
        for k in range(2):
            if pend_o[k] is not None:
                pend_o[k].wait()

    return sc_kernel


def kernel(input, mask, word_embeddings, position_embeddings):
    b, s = input.shape
    coef, mf, pstart, lpofs = _make_prep(b, s)(input, mask)
    del mf
    n = b * s
    out = _make_sc_gather(n)(
        input.reshape(n),
        pstart.reshape(n),
        lpofs.reshape(n),
        coef.reshape(n),
        word_embeddings,
        position_embeddings.reshape(-1),
    )
    return out.reshape(b, s, D)


# speedup vs baseline: 1.5209x; 1.5209x over previous
import dataclasses
import functools

import jax
import jax.numpy as jnp
from jax import lax
from jax.experimental import pallas as pl
from jax.experimental.pallas import tpu as pltpu
from jax.experimental.pallas import tpu_sc as plsc

MASK_ID = 3
PAD = 1
D = 768

NUM_CORES = 2
NUM_SUBCORES = 16
NW = NUM_CORES * NUM_SUBCORES
LANES = 16

W_CHUNK = 32

MASK_RATIO_TRAIN = 0.15 * 0.8


def _prep_body(inp_ref, mask_ref, coef_ref, mf_ref, pstart_ref, lpofs_ref):
    m = mask_ref[...]
    inp = inp_ref[...]
    b, s = m.shape
    ism = inp == MASK_ID
    c = m
    d = 1
    while d < s:
        shifted = jnp.concatenate(
            [jnp.zeros((b, d), jnp.int32), c[:, :-d]], axis=1
        )
        c = c + shifted
        d *= 2
    e = c - m
    r = jax.lax.broadcasted_iota(jnp.int32, (b, s), 1) & (W_CHUNK - 1)
    f = jnp.where(r == 0, e, 0)
    d = 1
    while d < W_CHUNK:
        shifted = jnp.concatenate(
            [jnp.zeros((b, d), jnp.int32), f[:, :-d]], axis=1
        )
        f = f + jnp.where(r >= d, shifted, 0)
        d *= 2
    arow = (f + 2) & ~7
    pstart_ref[...] = arow
    lpofs_ref[...] = jnp.where(m == 1, e + 2 - arow, 40)

    src_len = jnp.sum(m, axis=1, keepdims=True).astype(jnp.float32)
    n_mask = jnp.sum(ism.astype(jnp.int32), axis=1, keepdims=True).astype(
        jnp.float32
    )
    ratio = n_mask / src_len
    scale = jnp.minimum((1.0 - MASK_RATIO_TRAIN) / (1.0 - ratio), 4.0)
    coef_ref[...] = scale * m.astype(jnp.float32) * jnp.where(ism, 0.0, 1.0)
    mf_ref[...] = m.astype(jnp.float32)


def _make_prep(b, s):
    return pl.pallas_call(
        _prep_body,
        out_shape=(
            jax.ShapeDtypeStruct((b, s), jnp.float32),
            jax.ShapeDtypeStruct((b, s), jnp.float32),
            jax.ShapeDtypeStruct((b, s), jnp.int32),
            jax.ShapeDtypeStruct((b, s), jnp.int32),
        ),
    )


def _make_sc_gather(n_tokens):
    per_w = n_tokens // NW
    n_chunks = per_w // W_CHUNK
    P_ROWS = W_CHUNK + 8

    mesh = plsc.VectorSubcoreMesh(core_axis_name="c", subcore_axis_name="s")

    cp = pltpu.CompilerParams()
    if "needs_layout_passes" in pltpu.CompilerParams.__dataclass_fields__:
        cp = dataclasses.replace(cp, needs_layout_passes=False)

    @functools.partial(
        pl.kernel,
        out_type=jax.ShapeDtypeStruct((n_tokens, D), jnp.float32),
        mesh=mesh,
        compiler_params=cp,
        scratch_types=[
            pltpu.VMEM((per_w,), jnp.int32),
            pltpu.VMEM((per_w,), jnp.int32),
            pltpu.VMEM((per_w,), jnp.int32),
            pltpu.VMEM((per_w,), jnp.float32),
            pltpu.VMEM((W_CHUNK, D), jnp.float32),
            pltpu.VMEM((W_CHUNK, D), jnp.float32),
            pltpu.VMEM((P_ROWS + 1, D), jnp.float32),
            pltpu.VMEM((P_ROWS + 1, D), jnp.float32),
            pltpu.SemaphoreType.DMA,
            pltpu.SemaphoreType.DMA,
            pltpu.SemaphoreType.DMA,
            pltpu.SemaphoreType.DMA,
            pltpu.SemaphoreType.DMA,
        ],
    )
    def sc_kernel(
        idx_hbm, pstart_hbm, lpofs_hbm, coef_hbm, word_hbm,
        pos_hbm, out_hbm,
        idx_v, pstart_v, lpofs_v, coef_v, wb0, wb1, pb0, pb1,
        lsem, gs0, gs1, os0, os1,
    ):
        wid = lax.axis_index("s") * NUM_CORES + lax.axis_index("c")
        base = wid * per_w
        brow = wid // 8
        bcol = (wid % 8) * per_w
        sml = (
            pltpu.async_copy(
                idx_hbm.at[brow, pl.ds(bcol, per_w)], idx_v, lsem
            ),
            pltpu.async_copy(
                pstart_hbm.at[brow, pl.ds(bcol, per_w)], pstart_v, lsem
            ),
            pltpu.async_copy(
                lpofs_hbm.at[brow, pl.ds(bcol, per_w)], lpofs_v, lsem
            ),
            pltpu.async_copy(
                coef_hbm.at[brow, pl.ds(bcol, per_w)], coef_v, lsem
            ),
        )
        zero16 = jnp.zeros((LANES,), jnp.float32)
        for c0 in range(0, D, LANES):
            pb0.at[P_ROWS, pl.ds(c0, LANES)][...] = zero16
            pb1.at[P_ROWS, pl.ds(c0, LANES)][...] = zero16
        for c_ in sml:
            c_.wait()

        wb = (wb0, wb1)
        pb = (pb0, pb1)
        gs = (gs0, gs1)
        osem = (os0, os1)
        pend_g = [None, None]
        pend_o = [None, None]

        def issue_fetch(j):
            k = j % 2
            t0 = j * W_CHUNK
            cw = pltpu.async_copy(
                word_hbm.at[idx_v.at[pl.ds(t0, W_CHUNK)]], wb[k], gs[k]
            )
            start = pl.multiple_of(pstart_v[pl.ds(t0, LANES)][0], 8)
            cpos = pltpu.async_copy(
                pos_hbm.at[pl.ds(start, P_ROWS)],
                pb[k].at[pl.ds(0, P_ROWS)],
                gs[k],
            )
            pend_g[k] = (cw, cpos)

        issue_fetch(0)
        for j in range(n_chunks):
            k = j % 2
            if j + 1 < n_chunks:
                if pend_o[1 - k] is not None:
                    pend_o[1 - k].wait()
                    pend_o[1 - k] = None
                issue_fetch(j + 1)
            for c_ in pend_g[k]:
                c_.wait()
            pend_g[k] = None

            t0 = j * W_CHUNK

            rows = []
            for g in range(W_CHUNK // LANES):
                tg = t0 + g * LANES
                lp16 = lpofs_v[pl.ds(tg, LANES)]
                co16 = coef_v[pl.ds(tg, LANES)]
                for r in range(LANES):
                    rows.append((g * LANES + r, lp16[r], co16[r]))

            @plsc.parallel_loop(0, D, LANES)
            def _(c0):
                for tok, lp, co in rows:
                    w = wb[k].at[tok, pl.ds(c0, LANES)][...]
                    p = pb[k].at[lp, pl.ds(c0, LANES)][...]
                    wb[k].at[tok, pl.ds(c0, LANES)][...] = w * co + p

            pend_o[k] = pltpu.async_copy(
                wb[k], out_hbm.at[pl.ds(base + t0, W_CHUNK)], osem[k]
            )
        for k in range(2):
            if pend_o[k] is not None:
                pend_o[k].wait()

    return sc_kernel


def kernel(input, mask, word_embeddings, position_embeddings):
    b, s = input.shape
    coef, mf, pstart, lpofs = _make_prep(b, s)(input, mask)
    del mf
    n = b * s
    out = _make_sc_gather(n)(
        input,
        pstart,
        lpofs,
        coef,
        word_embeddings,
        position_embeddings,
    )
    return out.reshape(b, s, D)
